# baseline (speedup 1.0000x reference)
import functools

import jax
import jax.numpy as jnp
from jax import lax
from jax.experimental import pallas as pl
from jax.experimental.pallas import tpu as pltpu
from jax.experimental.pallas import tpu_sc as plsc

B = 16384
D = 128
L = 16
NSUB = D // L

NC = 2
NS = 16
NW = NC * NS
BPW = B // NW

CH = 64
NCH = BPW // CH


def _transh_body(head_hbm, rel_hbm, tail_hbm, ent_hbm, rele_hbm, relh_hbm,
                 out_hbm, hidx, tidx, ridx, hbuf, tbuf, wbuf, rbuf, obuf,
                 gsem, osem):
    cid = lax.axis_index("c")
    sid = lax.axis_index("s")
    wid = sid * NC + cid
    base = wid * BPW

    pltpu.sync_copy(head_hbm.at[pl.ds(base, BPW)], hidx)
    pltpu.sync_copy(tail_hbm.at[pl.ds(base, BPW)], tidx)
    pltpu.sync_copy(rel_hbm.at[pl.ds(base, BPW)], ridx)

    def issue_gathers(c, p):
        isl = pl.ds(c * CH, CH)
        return (
            pltpu.async_copy(ent_hbm.at[hidx.at[isl]], hbuf.at[p], gsem.at[p]),
            pltpu.async_copy(ent_hbm.at[tidx.at[isl]], tbuf.at[p], gsem.at[p]),
            pltpu.async_copy(relh_hbm.at[ridx.at[isl]], wbuf.at[p], gsem.at[p]),
            pltpu.async_copy(rele_hbm.at[ridx.at[isl]], rbuf.at[p], gsem.at[p]),
        )

    def compute(p):
        def row(i, carry):
            acc = jnp.zeros((L,), jnp.float32)
            hmts = []
            ws = []
            for j in range(NSUB):
                csl = pl.ds(j * L, L)
                h = hbuf[p, i, csl]
                t = tbuf[p, i, csl]
                w = wbuf[p, i, csl]
                hmt = h - t
                acc = acc + hmt * w
                hmts.append(hmt)
                ws.append(w)
            d = jnp.sum(acc)
            for j in range(NSUB):
                csl = pl.ds(j * L, L)
                r = rbuf[p, i, csl]
                obuf[p, i, csl] = hmts[j] + r - ws[j] * d
            return carry

        lax.fori_loop(0, CH, row, 0, unroll=2)

    gh = [None, None]
    oh = [None, None]
    gh[0] = issue_gathers(0, 0)
    for c in range(NCH):
        p = c % 2
        if c + 1 < NCH:
            gh[1 - p] = issue_gathers(c + 1, 1 - p)
        for h in gh[p]:
            h.wait()
        if oh[p] is not None:
            oh[p].wait()
            oh[p] = None
        compute(p)
        oh[p] = pltpu.async_copy(
            obuf.at[p], out_hbm.at[pl.ds(base + c * CH, CH)], osem.at[p])
    for p in (0, 1):
        if oh[p] is not None:
            oh[p].wait()


_transh = functools.partial(
    pl.kernel,
    out_type=jax.ShapeDtypeStruct((B, D), jnp.float32),
    mesh=plsc.VectorSubcoreMesh(core_axis_name="c", subcore_axis_name="s"),
    compiler_params=pltpu.CompilerParams(needs_layout_passes=False),
    scratch_types=[
        pltpu.VMEM((BPW,), jnp.int32),
        pltpu.VMEM((BPW,), jnp.int32),
        pltpu.VMEM((BPW,), jnp.int32),
        pltpu.VMEM((2, CH, D), jnp.float32),
        pltpu.VMEM((2, CH, D), jnp.float32),
        pltpu.VMEM((2, CH, D), jnp.float32),
        pltpu.VMEM((2, CH, D), jnp.float32),
        pltpu.VMEM((2, CH, D), jnp.float32),
        pltpu.SemaphoreType.DMA((2,)),
        pltpu.SemaphoreType.DMA((2,)),
    ],
)(_transh_body)


def kernel(head, relation, tail, ent_emb, rel_emb, rel_hyper):
    return _transh(head, relation, tail, ent_emb, rel_emb, rel_hyper)
